# Initial kernel scaffold; baseline (speedup 1.0000x reference)
import jax
import jax.numpy as jnp
from jax.experimental import pallas as pl


def kernel(x, edge_index, W1, a1_src, a1_dst, b1, W2, a2_src, a2_dst, b2):
    raise NotImplementedError("write your pallas kernel here")


# speedup vs baseline: 42.5340x; 42.5340x over previous
import functools

import jax
import jax.numpy as jnp
from jax import lax
from jax.experimental import pallas as pl
from jax.experimental.pallas import tpu as pltpu
from jax.experimental.pallas import tpu_sc as plsc

N = 10000
IN = 128
EMB = 16
H1 = 8
SLOPE = 0.2

NC = 2
NS = 16
NW = NC * NS
K = 128
RPT = 313
NP = NW * RPT

_mesh = plsc.VectorSubcoreMesh(
    core_axis_name="c", subcore_axis_name="s", num_cores=NC, num_subcores=NS)


def _leaky(v):
    return jnp.where(v > 0, v, v * SLOPE)


_GDN = lax.GatherDimensionNumbers(
    offset_dims=(), collapsed_slice_dims=(0,), start_index_map=(0,))


def _permute(vec, idx16):
    return lax.gather(vec, idx16[:, None], _GDN, (1,),
                      mode=lax.GatherScatterMode.PROMISE_IN_BOUNDS)


def _tc1_body(x_ref, w1_ref, as_ref, ad_ref, h1_ref, als_ref, ald_ref):
    h = jnp.dot(x_ref[...], w1_ref[...], preferred_element_type=jnp.float32)
    h1_ref[...] = h
    als_ref[...] = jnp.dot(h, as_ref[...], preferred_element_type=jnp.float32)
    ald_ref[...] = jnp.dot(h, ad_ref[...], preferred_element_type=jnp.float32)


def _tc2_body(accm_ref, accd_ref, r_ref, b1_ref, w2_ref, as2_ref, ad2_ref,
              h2_ref, als2_ref, ald2_ref):
    m = accm_ref[0] + accm_ref[1]
    d = accd_ref[0] + accd_ref[1]
    den = jnp.dot(d, r_ref[...], preferred_element_type=jnp.float32)
    h = _leaky(m / (den + 1e-16) + b1_ref[...])
    h2 = jnp.dot(h, w2_ref[...], preferred_element_type=jnp.float32)
    h2_ref[...] = h2
    als2_ref[...] = jnp.dot(h2, as2_ref[...], preferred_element_type=jnp.float32)
    ald2_ref[...] = jnp.dot(h2, ad2_ref[...], preferred_element_type=jnp.float32)


def _tc3_body(accm_ref, accd_ref, c_ref, b2_ref, out_ref):
    m = accm_ref[0] + accm_ref[1]
    d = accd_ref[0] + accd_ref[1]
    den = jnp.dot(d, c_ref[...], preferred_element_type=jnp.float32)
    out_ref[...] = _leaky(m / (den + 1e-16) + b2_ref[...])


def _sc1_body(h1t, als, ald, srcp, dstp, zm, zd,
              accm_o, accd_o,
              accm_sp, accd_sp, src_v, dst_v, hrows, als_v, ald_v,
              alpha_v, msg_v, sem):
    cid = lax.axis_index("c")
    sid = lax.axis_index("s")
    wid = cid * NS + sid
    n_chunks = srcp.shape[0] // (NW * K)

    pltpu.sync_copy(zm, accm_sp.at[pl.ds(sid * RPT, RPT)])
    pltpu.sync_copy(zd, accd_sp.at[pl.ds(sid * RPT, RPT)])
    plsc.subcore_barrier()

    lane = lax.iota(jnp.int32, 16)
    head_mask = lane < H1
    base0 = wid * (n_chunks * K)

    def chunk(g, carry):
        base = base0 + g * K
        pltpu.sync_copy(srcp.at[pl.ds(base, K)], src_v)
        pltpu.sync_copy(dstp.at[pl.ds(base, K)], dst_v)
        cp1 = pltpu.async_copy(h1t.at[src_v], hrows, sem)
        cp2 = pltpu.async_copy(als.at[src_v], als_v, sem)
        cp3 = pltpu.async_copy(ald.at[dst_v], ald_v, sem)
        cp1.wait()
        cp2.wait()
        cp3.wait()

        def edge(e, c2):
            a = jnp.exp(_leaky(als_v[e] + ald_v[e]))
            a = jnp.where(head_mask, a, 0.0)
            alpha_v[e] = a
            for j in range(H1):
                bc = _permute(a, jnp.full((16,), j, jnp.int32))
                msg_v[e, pl.ds(16 * j, 16)] = bc * hrows[e, pl.ds(16 * j, 16)]
            return c2

        lax.fori_loop(0, K, edge, 0)
        pltpu.async_copy(msg_v, accm_sp.at[dst_v], sem, add=True).wait()
        pltpu.async_copy(alpha_v, accd_sp.at[dst_v], sem, add=True).wait()
        return carry

    lax.fori_loop(0, n_chunks, chunk, 0)
    plsc.subcore_barrier()
    pltpu.sync_copy(accm_sp.at[pl.ds(sid * RPT, RPT)],
                    accm_o.at[cid, pl.ds(sid * RPT, RPT)])
    pltpu.sync_copy(accd_sp.at[pl.ds(sid * RPT, RPT)],
                    accd_o.at[cid, pl.ds(sid * RPT, RPT)])


def _sc2_body(h2t, als2, ald2, srcp, dstp, zd,
              accm_o, accd_o,
              accm_sp, accd_sp, src_v, dst_v, hrows, als_v, ald_v,
              alpha_v, msg_v, sem):
    cid = lax.axis_index("c")
    sid = lax.axis_index("s")
    wid = cid * NS + sid
    n_chunks = srcp.shape[0] // (NW * K)

    pltpu.sync_copy(zd, accm_sp.at[pl.ds(sid * RPT, RPT)])
    pltpu.sync_copy(zd, accd_sp.at[pl.ds(sid * RPT, RPT)])
    plsc.subcore_barrier()

    lane = lax.iota(jnp.int32, 16)
    col_mask = lane < 1
    zero16 = jnp.zeros((16,), jnp.int32)
    base0 = wid * (n_chunks * K)

    def chunk(g, carry):
        base = base0 + g * K
        pltpu.sync_copy(srcp.at[pl.ds(base, K)], src_v)
        pltpu.sync_copy(dstp.at[pl.ds(base, K)], dst_v)
        cp1 = pltpu.async_copy(h2t.at[src_v], hrows, sem)
        cp2 = pltpu.async_copy(als2.at[src_v], als_v, sem)
        cp3 = pltpu.async_copy(ald2.at[dst_v], ald_v, sem)
        cp1.wait()
        cp2.wait()
        cp3.wait()

        def edge(e, c2):
            a = jnp.exp(_leaky(als_v[e] + ald_v[e]))
            a = jnp.where(col_mask, a, 0.0)
            alpha_v[e] = a
            bc = _permute(a, zero16)
            msg_v[e] = bc * hrows[e]
            return c2

        lax.fori_loop(0, K, edge, 0)
        pltpu.async_copy(msg_v, accm_sp.at[dst_v], sem, add=True).wait()
        pltpu.async_copy(alpha_v, accd_sp.at[dst_v], sem, add=True).wait()
        return carry

    lax.fori_loop(0, n_chunks, chunk, 0)
    plsc.subcore_barrier()
    pltpu.sync_copy(accm_sp.at[pl.ds(sid * RPT, RPT)],
                    accm_o.at[cid, pl.ds(sid * RPT, RPT)])
    pltpu.sync_copy(accd_sp.at[pl.ds(sid * RPT, RPT)],
                    accd_o.at[cid, pl.ds(sid * RPT, RPT)])


_SC_PARAMS = pltpu.CompilerParams(use_tc_tiling_on_sc=False)

_sc1 = functools.partial(
    pl.kernel,
    out_type=(jax.ShapeDtypeStruct((NC, NP, IN), jnp.float32),
              jax.ShapeDtypeStruct((NC, NP, 16), jnp.float32)),
    mesh=_mesh,
    compiler_params=_SC_PARAMS,
    scratch_types=[
        pltpu.VMEM_SHARED((NP, IN), jnp.float32),
        pltpu.VMEM_SHARED((NP, 16), jnp.float32),
        pltpu.VMEM((K,), jnp.int32),
        pltpu.VMEM((K,), jnp.int32),
        pltpu.VMEM((K, IN), jnp.float32),
        pltpu.VMEM((K, 16), jnp.float32),
        pltpu.VMEM((K, 16), jnp.float32),
        pltpu.VMEM((K, 16), jnp.float32),
        pltpu.VMEM((K, IN), jnp.float32),
        pltpu.SemaphoreType.DMA,
    ],
)(_sc1_body)

_sc2 = functools.partial(
    pl.kernel,
    out_type=(jax.ShapeDtypeStruct((NC, NP, EMB), jnp.float32),
              jax.ShapeDtypeStruct((NC, NP, 16), jnp.float32)),
    mesh=_mesh,
    compiler_params=_SC_PARAMS,
    scratch_types=[
        pltpu.VMEM_SHARED((NP, EMB), jnp.float32),
        pltpu.VMEM_SHARED((NP, 16), jnp.float32),
        pltpu.VMEM((K,), jnp.int32),
        pltpu.VMEM((K,), jnp.int32),
        pltpu.VMEM((K, EMB), jnp.float32),
        pltpu.VMEM((K, 16), jnp.float32),
        pltpu.VMEM((K, 16), jnp.float32),
        pltpu.VMEM((K, 16), jnp.float32),
        pltpu.VMEM((K, EMB), jnp.float32),
        pltpu.SemaphoreType.DMA,
    ],
)(_sc2_body)


def kernel(x, edge_index, W1, a1_src, a1_dst, b1, W2, a2_src, a2_dst, b2):
    n = x.shape[0]
    e = edge_index.shape[1]
    loop = jnp.arange(n, dtype=edge_index.dtype)
    e_tot = e + n
    e_pad = ((e_tot + NW * K - 1) // (NW * K)) * (NW * K)
    pad = e_pad - e_tot
    srcp = jnp.concatenate([edge_index[0], loop, jnp.zeros((pad,), edge_index.dtype)])
    dstp = jnp.concatenate([edge_index[1], loop, jnp.full((pad,), n, edge_index.dtype)])

    f32 = jnp.float32
    cc = jnp.arange(IN)
    hh = jnp.arange(16)
    a1s_flat = a1_src.reshape(-1)
    a1d_flat = a1_dst.reshape(-1)
    blockdiag = (cc[:, None] // EMB == hh[None, :]).astype(f32)
    A1s = blockdiag * a1s_flat[:, None]
    A1d = blockdiag * a1d_flat[:, None]
    R = (jnp.arange(IN)[None, :] // EMB == jnp.arange(16)[:, None]).astype(f32)
    A2s = jnp.zeros((EMB, 16), f32).at[:, 0].set(a2_src[0])
    A2d = jnp.zeros((EMB, 16), f32).at[:, 0].set(a2_dst[0])
    C = jnp.zeros((16, EMB), f32).at[0, :].set(1.0)

    xp = jnp.zeros((NP, IN), f32).at[:n].set(x)
    zm = jnp.zeros((RPT, IN), f32)
    zd = jnp.zeros((RPT, 16), f32)

    BN = 2504
    G = NP // BN

    h1t, als, ald = pl.pallas_call(
        _tc1_body,
        grid=(G,),
        in_specs=[
            pl.BlockSpec((BN, IN), lambda i: (i, 0)),
            pl.BlockSpec((IN, IN), lambda i: (0, 0)),
            pl.BlockSpec((IN, 16), lambda i: (0, 0)),
            pl.BlockSpec((IN, 16), lambda i: (0, 0)),
        ],
        out_specs=[
            pl.BlockSpec((BN, IN), lambda i: (i, 0)),
            pl.BlockSpec((BN, 16), lambda i: (i, 0)),
            pl.BlockSpec((BN, 16), lambda i: (i, 0)),
        ],
        out_shape=[
            jax.ShapeDtypeStruct((NP, IN), f32),
            jax.ShapeDtypeStruct((NP, 16), f32),
            jax.ShapeDtypeStruct((NP, 16), f32),
        ],
    )(xp, W1, A1s, A1d)

    accm1, accd1 = _sc1(h1t, als, ald, srcp, dstp, zm, zd)

    h2t, als2, ald2 = pl.pallas_call(
        _tc2_body,
        grid=(G,),
        in_specs=[
            pl.BlockSpec((NC, BN, IN), lambda i: (0, i, 0)),
            pl.BlockSpec((NC, BN, 16), lambda i: (0, i, 0)),
            pl.BlockSpec((16, IN), lambda i: (0, 0)),
            pl.BlockSpec((1, IN), lambda i: (0, 0)),
            pl.BlockSpec((IN, EMB), lambda i: (0, 0)),
            pl.BlockSpec((EMB, 16), lambda i: (0, 0)),
            pl.BlockSpec((EMB, 16), lambda i: (0, 0)),
        ],
        out_specs=[
            pl.BlockSpec((BN, EMB), lambda i: (i, 0)),
            pl.BlockSpec((BN, 16), lambda i: (i, 0)),
            pl.BlockSpec((BN, 16), lambda i: (i, 0)),
        ],
        out_shape=[
            jax.ShapeDtypeStruct((NP, EMB), f32),
            jax.ShapeDtypeStruct((NP, 16), f32),
            jax.ShapeDtypeStruct((NP, 16), f32),
        ],
    )(accm1, accd1, R, b1.reshape(1, IN), W2, A2s, A2d)

    accm2, accd2 = _sc2(h2t, als2, ald2, srcp, dstp, zd)

    BN3 = 1000
    out = pl.pallas_call(
        _tc3_body,
        grid=(n // BN3,),
        in_specs=[
            pl.BlockSpec((NC, BN3, EMB), lambda i: (0, i, 0)),
            pl.BlockSpec((NC, BN3, 16), lambda i: (0, i, 0)),
            pl.BlockSpec((16, EMB), lambda i: (0, 0)),
            pl.BlockSpec((1, EMB), lambda i: (0, 0)),
        ],
        out_specs=pl.BlockSpec((BN3, EMB), lambda i: (i, 0)),
        out_shape=jax.ShapeDtypeStruct((n, EMB), f32),
    )(accm2, accd2, C, b2.reshape(1, EMB))

    return out


# speedup vs baseline: 52.7110x; 1.2393x over previous
import functools

import jax
import jax.numpy as jnp
from jax import lax
from jax.experimental import pallas as pl
from jax.experimental.pallas import tpu as pltpu
from jax.experimental.pallas import tpu_sc as plsc

N = 10000
IN = 128
EMB = 16
H1 = 8
SLOPE = 0.2

NC = 2
NS = 16
NW = NC * NS
K = 64
RPT = 313
NP = NW * RPT

_mesh = plsc.VectorSubcoreMesh(
    core_axis_name="c", subcore_axis_name="s", num_cores=NC, num_subcores=NS)


def _leaky(v):
    return jnp.where(v > 0, v, v * SLOPE)


_GDN = lax.GatherDimensionNumbers(
    offset_dims=(), collapsed_slice_dims=(0,), start_index_map=(0,))


def _permute(vec, idx16):
    return lax.gather(vec, idx16[:, None], _GDN, (1,),
                      mode=lax.GatherScatterMode.PROMISE_IN_BOUNDS)


def _tc1_body(x_ref, w1_ref, as_ref, ad_ref, h1_ref, als_ref, ald_ref):
    h = jnp.dot(x_ref[...], w1_ref[...], preferred_element_type=jnp.float32)
    h1_ref[...] = h
    als_ref[...] = jnp.dot(h, as_ref[...], preferred_element_type=jnp.float32)
    ald_ref[...] = jnp.dot(h, ad_ref[...], preferred_element_type=jnp.float32)


def _tc2_body(accm_ref, accd_ref, r_ref, b1_ref, w2_ref, as2_ref, ad2_ref,
              h2_ref, als2_ref, ald2_ref):
    m = accm_ref[0] + accm_ref[1]
    d = accd_ref[0] + accd_ref[1]
    den = jnp.dot(d, r_ref[...], preferred_element_type=jnp.float32)
    h = _leaky(m / (den + 1e-16) + b1_ref[...])
    h2 = jnp.dot(h, w2_ref[...], preferred_element_type=jnp.float32)
    h2_ref[...] = h2
    als2_ref[...] = jnp.dot(h2, as2_ref[...], preferred_element_type=jnp.float32)
    ald2_ref[...] = jnp.dot(h2, ad2_ref[...], preferred_element_type=jnp.float32)


def _tc3_body(accm_ref, accd_ref, c_ref, b2_ref, out_ref):
    m = accm_ref[0] + accm_ref[1]
    d = accd_ref[0] + accd_ref[1]
    den = jnp.dot(d, c_ref[...], preferred_element_type=jnp.float32)
    out_ref[...] = _leaky(m / (den + 1e-16) + b2_ref[...])


def _sc_body_factory(H):
    D = 16 * H

    def body(ht, als, ald, srcp, dstp, zm, zd,
             accm_o, accd_o,
             accm_sp, accd_sp,
             src0, src1, dst0, dst1, dsts0, dsts1,
             h0, h1_, as0, as1, ad0, ad1, al0, al1, m0, m1,
             semg0, semg1, sems0, sems1):
        cid = lax.axis_index("c")
        sid = lax.axis_index("s")
        wid = cid * NS + sid
        cpt = (srcp.shape[0] - 2 * K) // (NW * K)

        pltpu.sync_copy(zm, accm_sp.at[pl.ds(sid * RPT, RPT)])
        pltpu.sync_copy(zd, accd_sp.at[pl.ds(sid * RPT, RPT)])
        plsc.subcore_barrier()

        lane = lax.iota(jnp.int32, 16)
        head_mask = lane < H
        base0 = wid * (cpt * K)

        bufs = ((src0, dst0, dsts0, h0, as0, ad0, al0, m0, semg0, sems0),
                (src1, dst1, dsts1, h1_, as1, ad1, al1, m1, semg1, sems1))

        def fetch(srcb, dstb, hb, asb, adb, semg, g):
            base = base0 + g * K
            pltpu.sync_copy(srcp.at[pl.ds(base, K)], srcb)
            pltpu.sync_copy(dstp.at[pl.ds(base, K)], dstb)
            pltpu.async_copy(ht.at[srcb], hb, semg)
            pltpu.async_copy(als.at[srcb], asb, semg)
            pltpu.async_copy(ald.at[dstb], adb, semg)

        def half(g, b, first):
            srcb, dstb, dstsb, hb, asb, adb, alb, mb, semg, sems = bufs[b]
            pltpu.make_async_copy(ht.at[srcb], hb, semg).wait()
            pltpu.make_async_copy(als.at[srcb], asb, semg).wait()
            pltpu.make_async_copy(ald.at[dstb], adb, semg).wait()
            if not first:
                pltpu.make_async_copy(mb, accm_sp.at[dstsb], sems).wait()
                pltpu.make_async_copy(alb, accd_sp.at[dstsb], sems).wait()

            def edge(e, c2):
                a = jnp.exp(_leaky(asb[e] + adb[e]))
                a = jnp.where(head_mask, a, 0.0)
                alb[e] = a
                for j in range(H):
                    bc = _permute(a, jnp.full((16,), j, jnp.int32))
                    mb[e, pl.ds(16 * j, 16)] = bc * hb[e, pl.ds(16 * j, 16)]
                return c2

            lax.fori_loop(0, K, edge, 0)
            for i in range(K // 16):
                dstsb[pl.ds(16 * i, 16)] = dstb[pl.ds(16 * i, 16)]
            pltpu.async_copy(mb, accm_sp.at[dstsb], sems, add=True)
            pltpu.async_copy(alb, accd_sp.at[dstsb], sems, add=True)
            fetch(srcb, dstb, hb, asb, adb, semg, g + 2)

        for b in (0, 1):
            srcb, dstb, _, hb, asb, adb, _, _, semg, _ = bufs[b]
            fetch(srcb, dstb, hb, asb, adb, semg, b)
        half(0, 0, True)
        half(1, 1, True)

        def pair(i2, carry):
            g = 2 * i2
            half(g, 0, False)
            half(g + 1, 1, False)
            return carry

        lax.fori_loop(1, cpt // 2, pair, 0)

        for b in (0, 1):
            srcb, dstb, dstsb, hb, asb, adb, alb, mb, semg, sems = bufs[b]
            pltpu.make_async_copy(mb, accm_sp.at[dstsb], sems).wait()
            pltpu.make_async_copy(alb, accd_sp.at[dstsb], sems).wait()
            pltpu.make_async_copy(ht.at[srcb], hb, semg).wait()
            pltpu.make_async_copy(als.at[srcb], asb, semg).wait()
            pltpu.make_async_copy(ald.at[dstb], adb, semg).wait()

        plsc.subcore_barrier()
        pltpu.sync_copy(accm_sp.at[pl.ds(sid * RPT, RPT)],
                        accm_o.at[cid, pl.ds(sid * RPT, RPT)])
        pltpu.sync_copy(accd_sp.at[pl.ds(sid * RPT, RPT)],
                        accd_o.at[cid, pl.ds(sid * RPT, RPT)])

    return body


_sc1_body = _sc_body_factory(H1)
_sc2_body = _sc_body_factory(1)


_SC_PARAMS = pltpu.CompilerParams(use_tc_tiling_on_sc=False)


def _sc_scratch(D):
    return [
        pltpu.VMEM_SHARED((NP, D), jnp.float32),
        pltpu.VMEM_SHARED((NP, 16), jnp.float32),
        pltpu.VMEM((K,), jnp.int32),
        pltpu.VMEM((K,), jnp.int32),
        pltpu.VMEM((K,), jnp.int32),
        pltpu.VMEM((K,), jnp.int32),
        pltpu.VMEM((K,), jnp.int32),
        pltpu.VMEM((K,), jnp.int32),
        pltpu.VMEM((K, D), jnp.float32),
        pltpu.VMEM((K, D), jnp.float32),
        pltpu.VMEM((K, 16), jnp.float32),
        pltpu.VMEM((K, 16), jnp.float32),
        pltpu.VMEM((K, 16), jnp.float32),
        pltpu.VMEM((K, 16), jnp.float32),
        pltpu.VMEM((K, 16), jnp.float32),
        pltpu.VMEM((K, 16), jnp.float32),
        pltpu.VMEM((K, D), jnp.float32),
        pltpu.VMEM((K, D), jnp.float32),
        pltpu.SemaphoreType.DMA,
        pltpu.SemaphoreType.DMA,
        pltpu.SemaphoreType.DMA,
        pltpu.SemaphoreType.DMA,
    ]


_sc1 = functools.partial(
    pl.kernel,
    out_type=(jax.ShapeDtypeStruct((NC, NP, IN), jnp.float32),
              jax.ShapeDtypeStruct((NC, NP, 16), jnp.float32)),
    mesh=_mesh,
    compiler_params=_SC_PARAMS,
    scratch_types=_sc_scratch(IN),
)(_sc1_body)

_sc2 = functools.partial(
    pl.kernel,
    out_type=(jax.ShapeDtypeStruct((NC, NP, EMB), jnp.float32),
              jax.ShapeDtypeStruct((NC, NP, 16), jnp.float32)),
    mesh=_mesh,
    compiler_params=_SC_PARAMS,
    scratch_types=_sc_scratch(EMB),
)(_sc2_body)


def kernel(x, edge_index, W1, a1_src, a1_dst, b1, W2, a2_src, a2_dst, b2):
    n = x.shape[0]
    e = edge_index.shape[1]
    loop = jnp.arange(n, dtype=edge_index.dtype)
    e_tot = e + n
    blk = NW * K * 2
    e_pad = ((e_tot + blk - 1) // blk) * blk
    pad = e_pad - e_tot + 2 * K
    srcp = jnp.concatenate([edge_index[0], loop, jnp.zeros((pad,), edge_index.dtype)])
    dstp = jnp.concatenate([edge_index[1], loop, jnp.full((pad,), n, edge_index.dtype)])

    f32 = jnp.float32
    cc = jnp.arange(IN)
    hh = jnp.arange(16)
    a1s_flat = a1_src.reshape(-1)
    a1d_flat = a1_dst.reshape(-1)
    blockdiag = (cc[:, None] // EMB == hh[None, :]).astype(f32)
    A1s = blockdiag * a1s_flat[:, None]
    A1d = blockdiag * a1d_flat[:, None]
    R = (jnp.arange(IN)[None, :] // EMB == jnp.arange(16)[:, None]).astype(f32)
    A2s = jnp.zeros((EMB, 16), f32).at[:, 0].set(a2_src[0])
    A2d = jnp.zeros((EMB, 16), f32).at[:, 0].set(a2_dst[0])
    C = jnp.zeros((16, EMB), f32).at[0, :].set(1.0)

    xp = jnp.zeros((NP, IN), f32).at[:n].set(x)
    zm = jnp.zeros((RPT, IN), f32)
    zd = jnp.zeros((RPT, 16), f32)

    BN = 2504
    G = NP // BN

    h1t, als, ald = pl.pallas_call(
        _tc1_body,
        grid=(G,),
        in_specs=[
            pl.BlockSpec((BN, IN), lambda i: (i, 0)),
            pl.BlockSpec((IN, IN), lambda i: (0, 0)),
            pl.BlockSpec((IN, 16), lambda i: (0, 0)),
            pl.BlockSpec((IN, 16), lambda i: (0, 0)),
        ],
        out_specs=[
            pl.BlockSpec((BN, IN), lambda i: (i, 0)),
            pl.BlockSpec((BN, 16), lambda i: (i, 0)),
            pl.BlockSpec((BN, 16), lambda i: (i, 0)),
        ],
        out_shape=[
            jax.ShapeDtypeStruct((NP, IN), f32),
            jax.ShapeDtypeStruct((NP, 16), f32),
            jax.ShapeDtypeStruct((NP, 16), f32),
        ],
    )(xp, W1, A1s, A1d)

    accm1, accd1 = _sc1(h1t, als, ald, srcp, dstp, zm, zd)

    h2t, als2, ald2 = pl.pallas_call(
        _tc2_body,
        grid=(G,),
        in_specs=[
            pl.BlockSpec((NC, BN, IN), lambda i: (0, i, 0)),
            pl.BlockSpec((NC, BN, 16), lambda i: (0, i, 0)),
            pl.BlockSpec((16, IN), lambda i: (0, 0)),
            pl.BlockSpec((1, IN), lambda i: (0, 0)),
            pl.BlockSpec((IN, EMB), lambda i: (0, 0)),
            pl.BlockSpec((EMB, 16), lambda i: (0, 0)),
            pl.BlockSpec((EMB, 16), lambda i: (0, 0)),
        ],
        out_specs=[
            pl.BlockSpec((BN, EMB), lambda i: (i, 0)),
            pl.BlockSpec((BN, 16), lambda i: (i, 0)),
            pl.BlockSpec((BN, 16), lambda i: (i, 0)),
        ],
        out_shape=[
            jax.ShapeDtypeStruct((NP, EMB), f32),
            jax.ShapeDtypeStruct((NP, 16), f32),
            jax.ShapeDtypeStruct((NP, 16), f32),
        ],
    )(accm1, accd1, R, b1.reshape(1, IN), W2, A2s, A2d)

    accm2, accd2 = _sc2(h2t, als2, ald2, srcp, dstp, zd, zd)

    BN3 = 1000
    out = pl.pallas_call(
        _tc3_body,
        grid=(n // BN3,),
        in_specs=[
            pl.BlockSpec((NC, BN3, EMB), lambda i: (0, i, 0)),
            pl.BlockSpec((NC, BN3, 16), lambda i: (0, i, 0)),
            pl.BlockSpec((16, EMB), lambda i: (0, 0)),
            pl.BlockSpec((1, EMB), lambda i: (0, 0)),
        ],
        out_specs=pl.BlockSpec((BN3, EMB), lambda i: (i, 0)),
        out_shape=jax.ShapeDtypeStruct((n, EMB), f32),
    )(accm2, accd2, C, b2.reshape(1, EMB))

    return out
